# baseline (speedup 1.0000x reference)
import functools

import jax
import jax.numpy as jnp
from jax import lax
from jax.experimental import pallas as pl
from jax.experimental.pallas import tpu as pltpu
from jax.experimental.pallas import tpu_sc as plsc

B = 16384
L = 16
NW = 32
RPW = B // NW
G = RPW // L
XW = 66
CH = 128
NCH = RPW // CH

_f32 = jnp.float32
_i32 = jnp.int32


def _body(xf_hbm, cfu_hbm, cfi_hbm, nnu_hbm, nni_hbm, nnW_hbm, icW_hbm,
          ucW_hbm, par_hbm, out_hbm,
          x_v, ug_v, us_v, ig_v, is_v, cfu_b, cfi_b, nnu_b, nni_b, out_v,
          par_v, nnW_v, icW_v, ucW_v, sem):
    cid = lax.axis_index("c")
    sid = lax.axis_index("s")
    wid = cid * 16 + sid
    base = wid * RPW

    pltpu.sync_copy(par_hbm, par_v)
    pltpu.sync_copy(nnW_hbm, nnW_v)
    pltpu.sync_copy(icW_hbm, icW_v)
    pltpu.sync_copy(ucW_hbm, ucW_v)
    pltpu.sync_copy(xf_hbm.at[pl.ds(base * XW, RPW * XW)], x_v)

    lanes = lax.iota(_i32, L)

    def build(g, carry):
        flat = (g * L + lanes) * XW
        u = plsc.load_gather(x_v, [flat]).astype(_i32)
        i = plsc.load_gather(x_v, [flat + 1]).astype(_i32)
        sl = pl.ds(g * L, L)
        ug_v[sl] = jnp.right_shift(u, 3)
        us_v[sl] = jnp.bitwise_and(u, 7) * L
        ig_v[sl] = jnp.right_shift(i, 3)
        is_v[sl] = jnp.bitwise_and(i, 7) * L
        return carry
    lax.fori_loop(0, G, build, 0)

    pa_nn = par_v[0, :]
    pa_ic = par_v[1, :]
    pa_uc = par_v[2, :]
    pa_ab = par_v[3, :]
    wnnu = jnp.zeros((L,), _f32)
    wnni = jnp.zeros((L,), _f32)
    wic0 = jnp.zeros((L,), _f32)
    wic1 = jnp.zeros((L,), _f32)
    wuc0 = jnp.zeros((L,), _f32)
    wuc1 = jnp.zeros((L,), _f32)
    for k in range(16):
        s_nn = pa_nn[k]
        wnnu = wnnu + s_nn * nnW_v[k, pl.ds(0, L)]
        wnni = wnni + s_nn * nnW_v[k, pl.ds(L, L)]
        s_ic = pa_ic[k]
        wic0 = wic0 + s_ic * icW_v[k, pl.ds(0, L)]
        wic1 = wic1 + s_ic * icW_v[k, pl.ds(L, L)]
        s_uc = pa_uc[k]
        wuc0 = wuc0 + s_uc * ucW_v[k, pl.ds(0, L)]
        wuc1 = wuc1 + s_uc * ucW_v[k, pl.ds(L, L)]
    wfeat = (wic0, wic1, wuc0, wuc1)

    alpha = pa_ab[0]
    bias = (pa_ab[1]
            + jnp.sum(pa_nn * par_v[4, :])
            + jnp.sum(pa_ic * par_v[5, :])
            + jnp.sum(pa_uc * par_v[6, :]))

    def feats(g, carry):
        flat = (g * L + lanes) * XW + 2
        acc = bias + jnp.zeros((L,), _f32)
        for c in range(4):
            for dd in range(16):
                d = c * 16 + dd
                v = plsc.load_gather(x_v, [flat + d])
                acc = acc + wfeat[c][dd] * v
        out_v[pl.ds(g * L, L)] = acc
        return carry
    lax.fori_loop(0, G, feats, 0)

    def chunk(ch, carry):
        cbase = ch * CH
        isl = pl.ds(cbase, CH)
        d1 = pltpu.async_copy(cfu_hbm.at[ug_v.at[isl]], cfu_b, sem)
        d2 = pltpu.async_copy(cfi_hbm.at[ig_v.at[isl]], cfi_b, sem)
        d3 = pltpu.async_copy(nnu_hbm.at[ug_v.at[isl]], nnu_b, sem)
        d4 = pltpu.async_copy(nni_hbm.at[ig_v.at[isl]], nni_b, sem)
        d1.wait()
        d2.wait()
        d3.wait()
        d4.wait()
        for g2 in range(CH // L):
            rl = g2 * L + lanes
            sl = pl.ds(cbase + g2 * L, L)
            usub = us_v[sl]
            isub = is_v[sl]
            acc = out_v[sl]
            cfacc = jnp.zeros((L,), _f32)
            for d in range(D_EMB):
                cu = plsc.load_gather(cfu_b, [rl, usub + d])
                ci = plsc.load_gather(cfi_b, [rl, isub + d])
                cfacc = cfacc + cu * ci
                nu = plsc.load_gather(nnu_b, [rl, usub + d])
                acc = acc + wnnu[d] * nu
                ni = plsc.load_gather(nni_b, [rl, isub + d])
                acc = acc + wnni[d] * ni
            out_v[sl] = acc + alpha * cfacc
        return carry
    lax.fori_loop(0, NCH, chunk, 0)

    pltpu.sync_copy(out_v, out_hbm.at[pl.ds(base, RPW)])


D_EMB = 16

_sc_call = functools.partial(
    pl.kernel,
    out_type=jax.ShapeDtypeStruct((B,), _f32),
    mesh=plsc.VectorSubcoreMesh(core_axis_name="c", subcore_axis_name="s",
                                num_cores=2, num_subcores=16),
    compiler_params=pltpu.CompilerParams(needs_layout_passes=False,
                                         use_tc_tiling_on_sc=False),
    scratch_types=[
        pltpu.VMEM((RPW * XW,), _f32),
        pltpu.VMEM((RPW,), _i32),
        pltpu.VMEM((RPW,), _i32),
        pltpu.VMEM((RPW,), _i32),
        pltpu.VMEM((RPW,), _i32),
        pltpu.VMEM((CH, 128), _f32),
        pltpu.VMEM((CH, 128), _f32),
        pltpu.VMEM((CH, 128), _f32),
        pltpu.VMEM((CH, 128), _f32),
        pltpu.VMEM((RPW,), _f32),
        pltpu.VMEM((8, 16), _f32),
        pltpu.VMEM((16, 32), _f32),
        pltpu.VMEM((16, 32), _f32),
        pltpu.VMEM((16, 32), _f32),
        pltpu.SemaphoreType.DMA,
    ],
)(_body)


def kernel(x, cf_user_emb, cf_item_emb, nn_user_emb, nn_item_emb, nn_fc_W,
           nn_fc_b, ic_W, ic_b, uc_W, uc_b, fc_W, fc_b,
           item_context_features_in, user_context_features_in):
    row3 = jnp.concatenate([fc_W[0, 0:1], fc_b, jnp.zeros((14,), _f32)])
    params = jnp.stack([
        fc_W[0, 1:17], fc_W[0, 17:33], fc_W[0, 33:49], row3,
        nn_fc_b, ic_b, uc_b, jnp.zeros((16,), _f32),
    ])
    out = _sc_call(x.reshape(-1),
                   cf_user_emb.reshape(-1, 128), cf_item_emb.reshape(-1, 128),
                   nn_user_emb.reshape(-1, 128), nn_item_emb.reshape(-1, 128),
                   nn_fc_W, ic_W, uc_W, params)
    return out[:, None]


# speedup vs baseline: 1.0295x; 1.0295x over previous
import functools

import jax
import jax.numpy as jnp
from jax import lax
from jax.experimental import pallas as pl
from jax.experimental.pallas import tpu as pltpu
from jax.experimental.pallas import tpu_sc as plsc

B = 16384
L = 16
NW = 32
RPW = B // NW
G = RPW // L
XW = 66
D = 16

_f32 = jnp.float32
_i32 = jnp.int32


def _body(xT_hbm, cfu_hbm, cfi_hbm, nnu_hbm, nni_hbm, nnW_hbm, icW_hbm,
          ucW_hbm, par_hbm, out_hbm,
          xT_v, ug_v, ig_v, cfu_b, cfi_b, nnu_b, nni_b, out_v,
          par_v, nnW_v, icW_v, ucW_v, sem):
    cid = lax.axis_index("c")
    sid = lax.axis_index("s")
    wid = cid * 16 + sid
    base = wid * RPW

    pltpu.sync_copy(par_hbm, par_v)
    pltpu.sync_copy(nnW_hbm, nnW_v)
    pltpu.sync_copy(icW_hbm, icW_v)
    pltpu.sync_copy(ucW_hbm, ucW_v)
    pltpu.sync_copy(xT_hbm.at[:, pl.ds(base, RPW)], xT_v)

    lanes = lax.iota(_i32, L)

    def build(g, carry):
        sl = pl.ds(g * L, L)
        ug_v[sl] = xT_v[0, sl].astype(_i32)
        ig_v[sl] = xT_v[1, sl].astype(_i32)
        return carry
    lax.fori_loop(0, G, build, 0)

    c1 = pltpu.async_copy(cfu_hbm.at[ug_v], cfu_b, sem)
    c2 = pltpu.async_copy(cfi_hbm.at[ig_v], cfi_b, sem)
    c3 = pltpu.async_copy(nnu_hbm.at[ug_v], nnu_b, sem)
    c4 = pltpu.async_copy(nni_hbm.at[ig_v], nni_b, sem)

    pa_nn = par_v[0, :]
    pa_ic = par_v[1, :]
    pa_uc = par_v[2, :]
    pa_ab = par_v[3, :]
    wnnu = jnp.zeros((L,), _f32)
    wnni = jnp.zeros((L,), _f32)
    wic0 = jnp.zeros((L,), _f32)
    wic1 = jnp.zeros((L,), _f32)
    wuc0 = jnp.zeros((L,), _f32)
    wuc1 = jnp.zeros((L,), _f32)
    for k in range(16):
        s_nn = pa_nn[k]
        wnnu = wnnu + s_nn * nnW_v[k, pl.ds(0, L)]
        wnni = wnni + s_nn * nnW_v[k, pl.ds(L, L)]
        s_ic = pa_ic[k]
        wic0 = wic0 + s_ic * icW_v[k, pl.ds(0, L)]
        wic1 = wic1 + s_ic * icW_v[k, pl.ds(L, L)]
        s_uc = pa_uc[k]
        wuc0 = wuc0 + s_uc * ucW_v[k, pl.ds(0, L)]
        wuc1 = wuc1 + s_uc * ucW_v[k, pl.ds(L, L)]
    wfeat = (wic0, wic1, wuc0, wuc1)

    alpha = pa_ab[0]
    bias = (pa_ab[1]
            + jnp.sum(pa_nn * par_v[4, :])
            + jnp.sum(pa_ic * par_v[5, :])
            + jnp.sum(pa_uc * par_v[6, :]))

    def feats(g, carry):
        sl = pl.ds(g * L, L)
        acc = bias + jnp.zeros((L,), _f32)
        for c in range(4):
            for dd in range(16):
                d = c * 16 + dd
                acc = acc + wfeat[c][dd] * xT_v[2 + d, sl]
        out_v[sl] = acc
        return carry
    lax.fori_loop(0, G, feats, 0)

    c1.wait()
    c2.wait()
    c3.wait()
    c4.wait()

    def emb(g, carry):
        row = g * L + lanes
        sl = pl.ds(g * L, L)
        acc = out_v[sl]
        cfacc = jnp.zeros((L,), _f32)
        for d in range(D_EMB):
            col = jnp.zeros((L,), _i32) + d
            cu = plsc.load_gather(cfu_b, [row, col])
            ci = plsc.load_gather(cfi_b, [row, col])
            cfacc = cfacc + cu * ci
            nu = plsc.load_gather(nnu_b, [row, col])
            acc = acc + wnnu[d] * nu
            ni = plsc.load_gather(nni_b, [row, col])
            acc = acc + wnni[d] * ni
        out_v[sl] = acc + alpha * cfacc
        return carry
    lax.fori_loop(0, G, emb, 0)

    pltpu.sync_copy(out_v, out_hbm.at[pl.ds(base, RPW)])


D_EMB = 16

_sc_call = functools.partial(
    pl.kernel,
    out_type=jax.ShapeDtypeStruct((B,), _f32),
    mesh=plsc.VectorSubcoreMesh(core_axis_name="c", subcore_axis_name="s",
                                num_cores=2, num_subcores=16),
    compiler_params=pltpu.CompilerParams(needs_layout_passes=False,
                                         use_tc_tiling_on_sc=False),
    scratch_types=[
        pltpu.VMEM((XW, RPW), _f32),
        pltpu.VMEM((RPW,), _i32),
        pltpu.VMEM((RPW,), _i32),
        pltpu.VMEM((RPW, D), _f32),
        pltpu.VMEM((RPW, D), _f32),
        pltpu.VMEM((RPW, D), _f32),
        pltpu.VMEM((RPW, D), _f32),
        pltpu.VMEM((RPW,), _f32),
        pltpu.VMEM((8, 16), _f32),
        pltpu.VMEM((16, 32), _f32),
        pltpu.VMEM((16, 32), _f32),
        pltpu.VMEM((16, 32), _f32),
        pltpu.SemaphoreType.DMA,
    ],
)(_body)


def kernel(x, cf_user_emb, cf_item_emb, nn_user_emb, nn_item_emb, nn_fc_W,
           nn_fc_b, ic_W, ic_b, uc_W, uc_b, fc_W, fc_b,
           item_context_features_in, user_context_features_in):
    row3 = jnp.concatenate([fc_W[0, 0:1], fc_b, jnp.zeros((14,), _f32)])
    params = jnp.stack([
        fc_W[0, 1:17], fc_W[0, 17:33], fc_W[0, 33:49], row3,
        nn_fc_b, ic_b, uc_b, jnp.zeros((16,), _f32),
    ])
    out = _sc_call(x.T, cf_user_emb, cf_item_emb, nn_user_emb, nn_item_emb,
                   nn_fc_W, ic_W, uc_W, params)
    return out[:, None]


# speedup vs baseline: 4.1867x; 4.0669x over previous
import functools

import jax
import jax.numpy as jnp
from jax import lax
from jax.experimental import pallas as pl
from jax.experimental.pallas import tpu as pltpu
from jax.experimental.pallas import tpu_sc as plsc

B = 16384
L = 16
NW = 32
RPW = B // NW
G = RPW // L
XW = 66
D = 16
NU = 1000000
NI = 100000

CHU = 2048
NFULL = NU // CHU
TAIL0 = NFULL * CHU
TAIL1 = TAIL0 + 512
BUFW = D * CHU

CB = 64
NCB = RPW // CB
UROWS = NU // L

_f32 = jnp.float32
_i32 = jnp.int32


def _detile_body(cfuT_hbm, nnuT_hbm, tseg_cfu, tseg_nnu, cfu_f, nnu_f, buf,
                 tbuf, semr, semw0, semw1):
    cid = lax.axis_index("c")
    sid = lax.axis_index("s")
    wid = cid * 16 + sid
    sems = (semw0, semw1)

    def do_table(tT, tf, tseg):
        def outer(i, carry):
            for b2 in range(2):
                c = (i * 2 + b2) * NW + wid
                u0 = c * CHU

                @pl.when(jnp.logical_and(c >= 2 * NW, c < NFULL))
                def _drain():
                    pltpu.make_async_copy(
                        tf.at[pl.ds(0, BUFW)],
                        buf.at[pl.ds(b2 * BUFW, BUFW)], sems[b2]).wait()

                @pl.when(c < NFULL)
                def _work():
                    rds = []
                    for d in range(D):
                        rds.append(pltpu.async_copy(
                            tT.at[d, pl.ds(u0, CHU)],
                            buf.at[pl.ds(b2 * BUFW + d * CHU, CHU)], semr))
                    for r in rds:
                        r.wait()
                    for d in range(D):
                        pltpu.async_copy(
                            buf.at[pl.ds(b2 * BUFW + d * CHU, CHU)],
                            tf.at[pl.ds(d * NU + u0, CHU)], sems[b2])
            return carry
        lax.fori_loop(0, (NFULL + 2 * NW - 1) // (2 * NW), outer, 0)
        for b2 in range(2):
            pltpu.make_async_copy(tf.at[pl.ds(0, BUFW)],
                                  buf.at[pl.ds(b2 * BUFW, BUFW)],
                                  sems[b2]).wait()

        @pl.when(wid == 0)
        def _tail0():
            rds = [pltpu.async_copy(tT.at[d, pl.ds(TAIL0, 512)],
                                    tbuf.at[pl.ds(d * 512, 512)], semr)
                   for d in range(D)]
            for r in rds:
                r.wait()
            wrs = [pltpu.async_copy(tbuf.at[pl.ds(d * 512, 512)],
                                    tf.at[pl.ds(d * NU + TAIL0, 512)], semr)
                   for d in range(D)]
            for w in wrs:
                w.wait()

        @pl.when(wid == 1)
        def _tail1():
            rds = [pltpu.async_copy(tseg.at[pl.ds(d * 64, 64)],
                                    tbuf.at[pl.ds(d * 64, 64)], semr)
                   for d in range(D)]
            for r in rds:
                r.wait()
            wrs = [pltpu.async_copy(tbuf.at[pl.ds(d * 64, 64)],
                                    tf.at[pl.ds(d * NU + TAIL1, 64)], semr)
                   for d in range(D)]
            for w in wrs:
                w.wait()

    do_table(cfuT_hbm, cfu_f, tseg_cfu)
    do_table(nnuT_hbm, nnu_f, tseg_nnu)


_detile = functools.partial(
    pl.kernel,
    out_type=(jax.ShapeDtypeStruct((NU * D,), _f32),
              jax.ShapeDtypeStruct((NU * D,), _f32)),
    mesh=plsc.VectorSubcoreMesh(core_axis_name="c", subcore_axis_name="s",
                                num_cores=2, num_subcores=16),
    compiler_params=pltpu.CompilerParams(use_tc_tiling_on_sc=True),
    scratch_types=[
        pltpu.VMEM((2 * BUFW,), _f32),
        pltpu.VMEM((D * 512,), _f32),
        pltpu.SemaphoreType.DMA,
        pltpu.SemaphoreType.DMA,
        pltpu.SemaphoreType.DMA,
    ],
)(_detile_body)


def _main_body(xT_hbm, cfu_hbm, cfi_hbm, nnu_hbm, nni_hbm, nnW_hbm, icW_hbm,
               ucW_hbm, par_hbm, out_hbm,
               xT_v, ug_v, us_v, ig_v, cfu_b, nnu_b, cfi_b, nni_b, out_v,
               par_v, nnW_v, icW_v, ucW_v, sem):
    cid = lax.axis_index("c")
    sid = lax.axis_index("s")
    wid = cid * 16 + sid
    base = wid * RPW

    pltpu.sync_copy(par_hbm, par_v)
    pltpu.sync_copy(nnW_hbm, nnW_v)
    pltpu.sync_copy(icW_hbm, icW_v)
    pltpu.sync_copy(ucW_hbm, ucW_v)
    pltpu.sync_copy(xT_hbm.at[:, pl.ds(base, RPW)], xT_v)

    lanes = lax.iota(_i32, L)

    def build(g, carry):
        sl = pl.ds(g * L, L)
        u = xT_v[0, sl].astype(_i32)
        i = xT_v[1, sl].astype(_i32)
        uq = jnp.right_shift(u, 4)
        for d in range(D):
            ug_v[d, sl] = uq + d * UROWS
        us_v[sl] = jnp.bitwise_and(u, 15)
        ig_v[sl] = i
        return carry
    lax.fori_loop(0, G, build, 0)

    ci1 = pltpu.async_copy(cfi_hbm.at[ig_v], cfi_b, sem)
    ci2 = pltpu.async_copy(nni_hbm.at[ig_v], nni_b, sem)

    pa_nn = par_v[0, :]
    pa_ic = par_v[1, :]
    pa_uc = par_v[2, :]
    pa_ab = par_v[3, :]
    wnnu = jnp.zeros((L,), _f32)
    wnni = jnp.zeros((L,), _f32)
    wic0 = jnp.zeros((L,), _f32)
    wic1 = jnp.zeros((L,), _f32)
    wuc0 = jnp.zeros((L,), _f32)
    wuc1 = jnp.zeros((L,), _f32)
    for k in range(16):
        s_nn = pa_nn[k]
        wnnu = wnnu + s_nn * nnW_v[k, pl.ds(0, L)]
        wnni = wnni + s_nn * nnW_v[k, pl.ds(L, L)]
        s_ic = pa_ic[k]
        wic0 = wic0 + s_ic * icW_v[k, pl.ds(0, L)]
        wic1 = wic1 + s_ic * icW_v[k, pl.ds(L, L)]
        s_uc = pa_uc[k]
        wuc0 = wuc0 + s_uc * ucW_v[k, pl.ds(0, L)]
        wuc1 = wuc1 + s_uc * ucW_v[k, pl.ds(L, L)]
    wfeat = (wic0, wic1, wuc0, wuc1)

    alpha = pa_ab[0]
    bias = (pa_ab[1]
            + jnp.sum(pa_nn * par_v[4, :])
            + jnp.sum(pa_ic * par_v[5, :])
            + jnp.sum(pa_uc * par_v[6, :]))

    def feats(g, carry):
        sl = pl.ds(g * L, L)
        acc = bias + jnp.zeros((L,), _f32)
        for c in range(4):
            for dd in range(16):
                d = c * 16 + dd
                acc = acc + wfeat[c][dd] * xT_v[2 + d, sl]
        out_v[sl] = acc
        return carry
    lax.fori_loop(0, G, feats, 0)

    ci1.wait()
    ci2.wait()

    def chunk(ch, carry):
        cb = ch * CB
        cps = []
        for d in range(D):
            isl = pl.ds(cb, CB)
            cps.append(pltpu.async_copy(
                cfu_hbm.at[ug_v.at[d, isl]], cfu_b.at[d], sem))
            cps.append(pltpu.async_copy(
                nnu_hbm.at[ug_v.at[d, isl]], nnu_b.at[d], sem))
        for c in cps:
            c.wait()
        for g2 in range(CB // L):
            rl = g2 * L + lanes
            sl = pl.ds(cb + g2 * L, L)
            usub = us_v[sl]
            acc = out_v[sl]
            cfacc = jnp.zeros((L,), _f32)
            for d in range(D):
                dcol = jnp.zeros((L,), _i32) + d
                cu = plsc.load_gather(cfu_b, [dcol, rl, usub])
                nu = plsc.load_gather(nnu_b, [dcol, rl, usub])
                ci = plsc.load_gather(cfi_b, [cb + rl, dcol])
                ni = plsc.load_gather(nni_b, [cb + rl, dcol])
                cfacc = cfacc + cu * ci
                acc = acc + wnnu[d] * nu + wnni[d] * ni
            out_v[sl] = acc + alpha * cfacc
        return carry
    lax.fori_loop(0, NCB, chunk, 0)

    pltpu.sync_copy(out_v, out_hbm.at[pl.ds(base, RPW)])


_main = functools.partial(
    pl.kernel,
    out_type=jax.ShapeDtypeStruct((B,), _f32),
    mesh=plsc.VectorSubcoreMesh(core_axis_name="c", subcore_axis_name="s",
                                num_cores=2, num_subcores=16),
    compiler_params=pltpu.CompilerParams(needs_layout_passes=False,
                                         use_tc_tiling_on_sc=False),
    scratch_types=[
        pltpu.VMEM((XW, RPW), _f32),
        pltpu.VMEM((D, RPW), _i32),
        pltpu.VMEM((RPW,), _i32),
        pltpu.VMEM((RPW,), _i32),
        pltpu.VMEM((D, CB, L), _f32),
        pltpu.VMEM((D, CB, L), _f32),
        pltpu.VMEM((RPW, D), _f32),
        pltpu.VMEM((RPW, D), _f32),
        pltpu.VMEM((RPW,), _f32),
        pltpu.VMEM((8, 16), _f32),
        pltpu.VMEM((16, 32), _f32),
        pltpu.VMEM((16, 32), _f32),
        pltpu.VMEM((16, 32), _f32),
        pltpu.SemaphoreType.DMA,
    ],
)(_main_body)


def kernel(x, cf_user_emb, cf_item_emb, nn_user_emb, nn_item_emb, nn_fc_W,
           nn_fc_b, ic_W, ic_b, uc_W, uc_b, fc_W, fc_b,
           item_context_features_in, user_context_features_in):
    row3 = jnp.concatenate([fc_W[0, 0:1], fc_b, jnp.zeros((14,), _f32)])
    params = jnp.stack([
        fc_W[0, 1:17], fc_W[0, 17:33], fc_W[0, 33:49], row3,
        nn_fc_b, ic_b, uc_b, jnp.zeros((16,), _f32),
    ])
    tseg_cfu = cf_user_emb[TAIL1:, :].T.reshape(-1)
    tseg_nnu = nn_user_emb[TAIL1:, :].T.reshape(-1)
    cfu_f, nnu_f = _detile(cf_user_emb.T, nn_user_emb.T, tseg_cfu, tseg_nnu)
    out = _main(x.T, cfu_f.reshape(NU, D), cf_item_emb,
                nnu_f.reshape(NU, D), nn_item_emb,
                nn_fc_W, ic_W, uc_W, params)
    return out[:, None]
